# baseline (speedup 1.0000x reference)
import jax
import jax.numpy as jnp
from jax.experimental import pallas as pl
from jax.experimental.pallas import tpu as pltpu

N_ROWS = 8192
L0 = 1024
L1 = 512
DE = 256
NE = 4
NC = 256
BLK = 1024
GRID = N_ROWS // BLK


def _sigmoid(v):
    return 0.5 * jnp.tanh(0.5 * v) + 0.5


def _body(h_ref, w1_ref, b1_ref, ua_ref, ub_ref, uct_ref, wc_ref, bc_ref,
          cb_ref, a_ref, logits_ref, probs_ref, misc_ref, p_acc, s_acc):
    i = pl.program_id(0)

    @pl.when(i == 0)
    def _init():
        p_acc[...] = jnp.zeros_like(p_acc)
        s_acc[0] = 0.0

    x = jax.lax.dot_general(
        h_ref[...].astype(jnp.bfloat16), w1_ref[...],
        (((1,), (1,)), ((), ())),
        preferred_element_type=jnp.float32)
    x = jnp.maximum(x + b1_ref[...], 0.0)
    xb = x.astype(jnp.bfloat16)

    att = jnp.zeros((1, BLK), jnp.float32)
    for e in range(NE):
        ya = jax.lax.dot_general(
            xb, ua_ref[e], (((1,), (0,)), ((), ())),
            preferred_element_type=jnp.float32)
        yb = jax.lax.dot_general(
            xb, ub_ref[e], (((1,), (0,)), ((), ())),
            preferred_element_type=jnp.float32)
        g = jnp.tanh(ya) * _sigmoid(yb)
        att += jax.lax.dot_general(
            uct_ref[e], g, (((1,), (1,)), ((), ())),
            preferred_element_type=jnp.float32)
    a_row = _sigmoid(att)
    a_ref[...] = a_row

    p_acc[...] += jax.lax.dot_general(
        a_row, x, (((1,), (0,)), ((), ())),
        preferred_element_type=jnp.float32)
    s_acc[0] += jnp.sum(a_row)

    @pl.when(i == GRID - 1)
    def _tail():
        m = p_acc[...] / s_acc[0]
        cb = cb_ref[...]
        mc = jax.lax.dot_general(
            cb, m, (((1,), (1,)), ((), ())),
            preferred_element_type=jnp.float32)
        cbsq = jnp.sum(cb * cb, axis=1, keepdims=True)
        msq = jnp.sum(m * m)
        dist = msq + cbsq + 2.0 * mc
        dmin = jnp.min(dist)
        iota_c = jax.lax.broadcasted_iota(jnp.int32, (NC, 1), 0)
        enc = jnp.min(jnp.where(dist == dmin, iota_c, NC))
        onehot = (jax.lax.broadcasted_iota(jnp.int32, (1, NC), 1) == enc
                  ).astype(jnp.float32)
        q = jax.lax.dot_general(
            onehot, cb, (((1,), (0,)), ((), ())),
            preferred_element_type=jnp.float32)
        vq_loss = 1.25 * jnp.sum((q - m) ** 2) / L1

        logits = jax.lax.dot_general(
            m, wc_ref[...], (((1,), (1,)), ((), ())),
            preferred_element_type=jnp.float32) + bc_ref[...]
        lmax = jnp.max(logits, axis=1, keepdims=True)
        e2 = jnp.exp(logits - lmax)
        probs = e2 / jnp.sum(e2, axis=1, keepdims=True)
        iota2 = jax.lax.broadcasted_iota(jnp.int32, (1, 2), 1)
        l0 = jnp.sum(jnp.where(iota2 == 0, logits, 0.0))
        l1 = jnp.sum(jnp.where(iota2 == 1, logits, 0.0))
        yhat = jnp.where(l1 > l0, 1.0, 0.0)
        logits_ref[...] = logits
        probs_ref[...] = probs
        misc_ref[...] = jnp.where(iota2 == 0, vq_loss, yhat)


def kernel(h, W1, b1, Ua, Ub, Uc, Wc, bc, codebook):
    uc_t = Uc.reshape(NE, 1, DE)
    b1r = b1.reshape(1, L1)
    bcr = bc.reshape(1, 2)
    w1b = W1.astype(jnp.bfloat16)
    uab = Ua.astype(jnp.bfloat16)
    ubb = Ub.astype(jnp.bfloat16)

    a_out, logits, probs, misc = pl.pallas_call(
        _body,
        grid=(GRID,),
        in_specs=[
            pl.BlockSpec((BLK, L0), lambda i: (i, 0)),
            pl.BlockSpec((L1, L0), lambda i: (0, 0)),
            pl.BlockSpec((1, L1), lambda i: (0, 0)),
            pl.BlockSpec((NE, L1, DE), lambda i: (0, 0, 0)),
            pl.BlockSpec((NE, L1, DE), lambda i: (0, 0, 0)),
            pl.BlockSpec((NE, 1, DE), lambda i: (0, 0, 0)),
            pl.BlockSpec((2, L1), lambda i: (0, 0)),
            pl.BlockSpec((1, 2), lambda i: (0, 0)),
            pl.BlockSpec((NC, L1), lambda i: (0, 0)),
        ],
        out_specs=[
            pl.BlockSpec((1, BLK), lambda i: (0, i)),
            pl.BlockSpec((1, 2), lambda i: (0, 0)),
            pl.BlockSpec((1, 2), lambda i: (0, 0)),
            pl.BlockSpec((1, 2), lambda i: (0, 0)),
        ],
        out_shape=[
            jax.ShapeDtypeStruct((1, N_ROWS), jnp.float32),
            jax.ShapeDtypeStruct((1, 2), jnp.float32),
            jax.ShapeDtypeStruct((1, 2), jnp.float32),
            jax.ShapeDtypeStruct((1, 2), jnp.float32),
        ],
        scratch_shapes=[
            pltpu.VMEM((1, L1), jnp.float32),
            pltpu.SMEM((1,), jnp.float32),
        ],
        compiler_params=pltpu.CompilerParams(
            dimension_semantics=("arbitrary",)),
    )(h, w1b, b1r, uab, ubb, uc_t, Wc, bcr, codebook)

    top_instance = logits
    y_probs = probs
    Y_prob = probs
    Y_hat = misc[:, 1:2].astype(jnp.int32)
    vq_loss = misc[0, 0]
    return (top_instance, Y_prob, Y_hat, vq_loss, y_probs, a_out)


# speedup vs baseline: 1.1550x; 1.1550x over previous
import jax
import jax.numpy as jnp
from jax.experimental import pallas as pl
from jax.experimental.pallas import tpu as pltpu

N_ROWS = 8192
L0 = 1024
L1 = 512
DE = 256
NE = 4
NC = 256
BLK = 2048
GRID = N_ROWS // BLK


def _sigmoid(v):
    return 0.5 * jnp.tanh(0.5 * v) + 0.5


def _body(h_ref, w1_ref, b1_ref, ua_ref, ub_ref, uct_ref, wc_ref, bc_ref,
          cb_ref, a_ref, logits_ref, probs_ref, misc_ref, p_acc, s_acc):
    i = pl.program_id(0)

    @pl.when(i == 0)
    def _init():
        p_acc[...] = jnp.zeros_like(p_acc)
        s_acc[0] = 0.0

    x = jax.lax.dot_general(
        h_ref[...].astype(jnp.bfloat16), w1_ref[...].astype(jnp.bfloat16),
        (((1,), (1,)), ((), ())),
        preferred_element_type=jnp.float32)
    x = jnp.maximum(x + b1_ref[...], 0.0)
    xb = x.astype(jnp.bfloat16)

    att = jnp.zeros((1, BLK), jnp.float32)
    for e in range(NE):
        ya = jax.lax.dot_general(
            xb, ua_ref[e].astype(jnp.bfloat16), (((1,), (0,)), ((), ())),
            preferred_element_type=jnp.float32)
        yb = jax.lax.dot_general(
            xb, ub_ref[e].astype(jnp.bfloat16), (((1,), (0,)), ((), ())),
            preferred_element_type=jnp.float32)
        g = jnp.tanh(ya) * _sigmoid(yb)
        att += jax.lax.dot_general(
            uct_ref[e], g, (((1,), (1,)), ((), ())),
            preferred_element_type=jnp.float32)
    a_row = _sigmoid(att)
    a_ref[...] = a_row

    p_acc[...] += jax.lax.dot_general(
        a_row, x, (((1,), (0,)), ((), ())),
        preferred_element_type=jnp.float32)
    s_acc[0] += jnp.sum(a_row)

    @pl.when(i == GRID - 1)
    def _tail():
        m = p_acc[...] / s_acc[0]
        cb = cb_ref[...]
        mc = jax.lax.dot_general(
            cb, m, (((1,), (1,)), ((), ())),
            preferred_element_type=jnp.float32)
        cbsq = jnp.sum(cb * cb, axis=1, keepdims=True)
        msq = jnp.sum(m * m)
        dist = msq + cbsq + 2.0 * mc
        dmin = jnp.min(dist)
        iota_c = jax.lax.broadcasted_iota(jnp.int32, (NC, 1), 0)
        enc = jnp.min(jnp.where(dist == dmin, iota_c, NC))
        onehot = (jax.lax.broadcasted_iota(jnp.int32, (1, NC), 1) == enc
                  ).astype(jnp.float32)
        q = jax.lax.dot_general(
            onehot, cb, (((1,), (0,)), ((), ())),
            preferred_element_type=jnp.float32)
        vq_loss = 1.25 * jnp.sum((q - m) ** 2) / L1

        logits = jax.lax.dot_general(
            m, wc_ref[...], (((1,), (1,)), ((), ())),
            preferred_element_type=jnp.float32) + bc_ref[...]
        lmax = jnp.max(logits, axis=1, keepdims=True)
        e2 = jnp.exp(logits - lmax)
        probs = e2 / jnp.sum(e2, axis=1, keepdims=True)
        iota2 = jax.lax.broadcasted_iota(jnp.int32, (1, 2), 1)
        l0 = jnp.sum(jnp.where(iota2 == 0, logits, 0.0))
        l1 = jnp.sum(jnp.where(iota2 == 1, logits, 0.0))
        yhat = jnp.where(l1 > l0, 1.0, 0.0)
        logits_ref[...] = logits
        probs_ref[...] = probs
        misc_ref[...] = jnp.where(iota2 == 0, vq_loss, yhat)


def kernel(h, W1, b1, Ua, Ub, Uc, Wc, bc, codebook):
    uc_t = Uc.reshape(NE, 1, DE)
    b1r = b1.reshape(1, L1)
    bcr = bc.reshape(1, 2)

    a_out, logits, probs, misc = pl.pallas_call(
        _body,
        grid=(GRID,),
        in_specs=[
            pl.BlockSpec((BLK, L0), lambda i: (i, 0)),
            pl.BlockSpec((L1, L0), lambda i: (0, 0)),
            pl.BlockSpec((1, L1), lambda i: (0, 0)),
            pl.BlockSpec((NE, L1, DE), lambda i: (0, 0, 0)),
            pl.BlockSpec((NE, L1, DE), lambda i: (0, 0, 0)),
            pl.BlockSpec((NE, 1, DE), lambda i: (0, 0, 0)),
            pl.BlockSpec((2, L1), lambda i: (0, 0)),
            pl.BlockSpec((1, 2), lambda i: (0, 0)),
            pl.BlockSpec((NC, L1), lambda i: (0, 0)),
        ],
        out_specs=[
            pl.BlockSpec((1, BLK), lambda i: (0, i)),
            pl.BlockSpec((1, 2), lambda i: (0, 0)),
            pl.BlockSpec((1, 2), lambda i: (0, 0)),
            pl.BlockSpec((1, 2), lambda i: (0, 0)),
        ],
        out_shape=[
            jax.ShapeDtypeStruct((1, N_ROWS), jnp.float32),
            jax.ShapeDtypeStruct((1, 2), jnp.float32),
            jax.ShapeDtypeStruct((1, 2), jnp.float32),
            jax.ShapeDtypeStruct((1, 2), jnp.float32),
        ],
        scratch_shapes=[
            pltpu.VMEM((1, L1), jnp.float32),
            pltpu.SMEM((1,), jnp.float32),
        ],
        compiler_params=pltpu.CompilerParams(
            dimension_semantics=("arbitrary",)),
    )(h, W1, b1r, Ua, Ub, uc_t, Wc, bcr, codebook)

    top_instance = logits
    y_probs = probs
    Y_prob = probs
    Y_hat = misc[:, 1:2].astype(jnp.int32)
    vq_loss = misc[0, 0]
    return (top_instance, Y_prob, Y_hat, vq_loss, y_probs, a_out)


# speedup vs baseline: 1.2365x; 1.0706x over previous
import jax
import jax.numpy as jnp
from jax.experimental import pallas as pl
from jax.experimental.pallas import tpu as pltpu

N_ROWS = 8192
L0 = 1024
L1 = 512
DE = 256
NE = 4
NC = 256
BLK = 2048
GRID = N_ROWS // BLK


def _sigmoid(v):
    return 0.5 * jnp.tanh(0.5 * v) + 0.5


def _body(h_ref, w1_ref, b1_ref, ua_ref, ub_ref, uct_ref, wc_ref, bc_ref,
          cb_ref, a_ref, logits_ref, probs_ref, vq_ref, yhat_ref,
          p_acc, s_acc):
    i = pl.program_id(0)

    @pl.when(i == 0)
    def _init():
        p_acc[...] = jnp.zeros_like(p_acc)
        s_acc[0] = 0.0

    x0 = jax.lax.dot_general(
        h_ref[...].astype(jnp.bfloat16), w1_ref[...].astype(jnp.bfloat16),
        (((1,), (1,)), ((), ())),
        preferred_element_type=jnp.float32)
    xb = jnp.maximum(x0 + b1_ref[...], 0.0).astype(jnp.bfloat16)

    att = jnp.zeros((1, BLK), jnp.float32)
    for e in range(NE):
        ya = jax.lax.dot_general(
            xb, ua_ref[e].astype(jnp.bfloat16), (((1,), (0,)), ((), ())),
            preferred_element_type=jnp.float32)
        yb = jax.lax.dot_general(
            xb, ub_ref[e].astype(jnp.bfloat16), (((1,), (0,)), ((), ())),
            preferred_element_type=jnp.float32)
        g = jnp.tanh(ya) * _sigmoid(yb)
        att += jax.lax.dot_general(
            uct_ref[e], g, (((1,), (1,)), ((), ())),
            preferred_element_type=jnp.float32)
    a_row = _sigmoid(att)
    a_ref[...] = a_row

    p_acc[...] += jax.lax.dot_general(
        a_row.astype(jnp.bfloat16), xb, (((1,), (0,)), ((), ())),
        preferred_element_type=jnp.float32)
    s_acc[0] += jnp.sum(a_row)

    @pl.when(i == GRID - 1)
    def _tail():
        m = p_acc[...] / s_acc[0]
        cb = cb_ref[...]
        mc = jax.lax.dot_general(
            cb, m, (((1,), (1,)), ((), ())),
            preferred_element_type=jnp.float32)
        cbsq = jnp.sum(cb * cb, axis=1, keepdims=True)
        msq = jnp.sum(m * m)
        dist = msq + cbsq + 2.0 * mc
        dmin = jnp.min(dist)
        iota_c = jax.lax.broadcasted_iota(jnp.int32, (NC, 1), 0)
        enc = jnp.min(jnp.where(dist == dmin, iota_c, NC))
        onehot = (jax.lax.broadcasted_iota(jnp.int32, (1, NC), 1) == enc
                  ).astype(jnp.float32)
        q = jax.lax.dot_general(
            onehot, cb, (((1,), (0,)), ((), ())),
            preferred_element_type=jnp.float32)
        vq_loss = 1.25 * jnp.sum((q - m) ** 2) / L1

        logits = jax.lax.dot_general(
            m, wc_ref[...], (((1,), (1,)), ((), ())),
            preferred_element_type=jnp.float32) + bc_ref[...]
        lmax = jnp.max(logits, axis=1, keepdims=True)
        e2 = jnp.exp(logits - lmax)
        probs = e2 / jnp.sum(e2, axis=1, keepdims=True)
        iota2 = jax.lax.broadcasted_iota(jnp.int32, (1, 2), 1)
        l0 = jnp.sum(jnp.where(iota2 == 0, logits, 0.0))
        l1 = jnp.sum(jnp.where(iota2 == 1, logits, 0.0))
        logits_ref[...] = logits
        probs_ref[...] = probs
        vq_ref[...] = jnp.broadcast_to(vq_loss, (1, 1))
        yhat_ref[...] = jnp.broadcast_to(l1 > l0, (1, 1)).astype(jnp.int32)


def kernel(h, W1, b1, Ua, Ub, Uc, Wc, bc, codebook):
    uc_t = Uc.reshape(NE, 1, DE)
    b1r = b1.reshape(1, L1)
    bcr = bc.reshape(1, 2)

    a_out, logits, probs, vqo, yhato = pl.pallas_call(
        _body,
        grid=(GRID,),
        in_specs=[
            pl.BlockSpec((BLK, L0), lambda i: (i, 0)),
            pl.BlockSpec((L1, L0), lambda i: (0, 0)),
            pl.BlockSpec((1, L1), lambda i: (0, 0)),
            pl.BlockSpec((NE, L1, DE), lambda i: (0, 0, 0)),
            pl.BlockSpec((NE, L1, DE), lambda i: (0, 0, 0)),
            pl.BlockSpec((NE, 1, DE), lambda i: (0, 0, 0)),
            pl.BlockSpec((2, L1), lambda i: (0, 0)),
            pl.BlockSpec((1, 2), lambda i: (0, 0)),
            pl.BlockSpec((NC, L1), lambda i: (0, 0)),
        ],
        out_specs=[
            pl.BlockSpec((1, BLK), lambda i: (0, i)),
            pl.BlockSpec((1, 2), lambda i: (0, 0)),
            pl.BlockSpec((1, 2), lambda i: (0, 0)),
            pl.BlockSpec((1, 1), lambda i: (0, 0)),
            pl.BlockSpec((1, 1), lambda i: (0, 0)),
        ],
        out_shape=[
            jax.ShapeDtypeStruct((1, N_ROWS), jnp.float32),
            jax.ShapeDtypeStruct((1, 2), jnp.float32),
            jax.ShapeDtypeStruct((1, 2), jnp.float32),
            jax.ShapeDtypeStruct((1, 1), jnp.float32),
            jax.ShapeDtypeStruct((1, 1), jnp.int32),
        ],
        scratch_shapes=[
            pltpu.VMEM((1, L1), jnp.float32),
            pltpu.SMEM((1,), jnp.float32),
        ],
        compiler_params=pltpu.CompilerParams(
            dimension_semantics=("arbitrary",)),
    )(h, W1, b1r, Ua, Ub, uc_t, Wc, bcr, codebook)

    top_instance = logits
    vq_loss = vqo.reshape(())
    return (top_instance, probs, yhato, vq_loss, probs, a_out)
